# baseline (speedup 1.0000x reference)
import functools

import jax
import jax.numpy as jnp
from jax import lax
from jax.experimental import pallas as pl
from jax.experimental.pallas import tpu as pltpu
from jax.experimental.pallas import tpu_sc as plsc

K = 8192
D = 256
IN_D = 768
B = 4096
BM = 1024

_PREC = None


def _enc_dist_kernel(x_ref, w_ref, b_ref, cb_ref, csq_ref, cbn_ref, idx_ref, csqn_ref):
    cbn = cbn_ref[...]
    csqn_ref[...] = jnp.sum(cbn * cbn, axis=1)
    r0 = (
        lax.dot_general(
            x_ref[...], w_ref[...], (((1,), (0,)), ((), ())), precision=_PREC
        )
        + b_ref[...]
    )
    _argmin_body(r0, cb_ref, csq_ref[...], idx_ref)


def _dist_csq_kernel(r_ref, cb_ref, csq_ref, cbn_ref, idx_ref, csqn_ref):
    cbn = cbn_ref[...]
    csqn_ref[...] = jnp.sum(cbn * cbn, axis=1)
    _argmin_body(r_ref[...], cb_ref, csq_ref[...].reshape(1, K), idx_ref)


def _dist_kernel(r_ref, cb_ref, csq_ref, idx_ref):
    _argmin_body(r_ref[...], cb_ref, csq_ref[...].reshape(1, K), idx_ref)


def _argmin_body(residual, cb_ref, csq, idx_ref):
    r_sq = jnp.sum(residual * residual, axis=1, keepdims=True)
    cross2 = lax.dot_general(
        residual + residual, cb_ref[...], (((1,), (1,)), ((), ())), precision=_PREC
    )
    dist = r_sq - cross2 + csq
    m = jnp.min(dist, axis=1, keepdims=True)
    lane = lax.broadcasted_iota(jnp.int32, (1, K), 1).astype(jnp.float32)
    idxf = jnp.min(jnp.where(dist == m, lane, float(K)), axis=1)
    idx_ref[...] = idxf.astype(jnp.int32)


def _decode_kernel(q0_ref, q1_ref, q2_ref, w_ref, b_ref, out_ref):
    rec = q0_ref[...] + q1_ref[...] + q2_ref[...]
    out_ref[...] = (
        lax.dot_general(rec, w_ref[...], (((1,), (0,)), ((), ())), precision=_PREC)
        + b_ref[...]
    )


_KC = K // (B // BM)


def _tc_enc_dist(x, enc_W, enc_b, cb, csq, cb_next):
    grid = (B // BM,)
    return pl.pallas_call(
        _enc_dist_kernel,
        grid=grid,
        in_specs=[
            pl.BlockSpec((BM, IN_D), lambda i: (i, 0)),
            pl.BlockSpec((IN_D, D), lambda i: (0, 0)),
            pl.BlockSpec((1, D), lambda i: (0, 0)),
            pl.BlockSpec((K, D), lambda i: (0, 0)),
            pl.BlockSpec((1, K), lambda i: (0, 0)),
            pl.BlockSpec((_KC, D), lambda i: (i, 0)),
        ],
        out_specs=[
            pl.BlockSpec((BM,), lambda i: (i,)),
            pl.BlockSpec((_KC,), lambda i: (i,)),
        ],
        out_shape=[
            jax.ShapeDtypeStruct((B,), jnp.int32),
            jax.ShapeDtypeStruct((K,), jnp.float32),
        ],
        compiler_params=pltpu.CompilerParams(
            dimension_semantics=("arbitrary",),
        ),
    )(x, enc_W, enc_b, cb, csq, cb_next)


def _tc_dist_csq(r, cb, csq, cb_next):
    grid = (B // BM,)
    return pl.pallas_call(
        _dist_csq_kernel,
        grid=grid,
        in_specs=[
            pl.BlockSpec((BM, D), lambda i: (i, 0)),
            pl.BlockSpec((K, D), lambda i: (0, 0)),
            pl.BlockSpec((K,), lambda i: (0,)),
            pl.BlockSpec((_KC, D), lambda i: (i, 0)),
        ],
        out_specs=[
            pl.BlockSpec((BM,), lambda i: (i,)),
            pl.BlockSpec((_KC,), lambda i: (i,)),
        ],
        out_shape=[
            jax.ShapeDtypeStruct((B,), jnp.int32),
            jax.ShapeDtypeStruct((K,), jnp.float32),
        ],
        compiler_params=pltpu.CompilerParams(
            dimension_semantics=("arbitrary",),
        ),
    )(r, cb, csq, cb_next)


def _tc_dist(r, cb, csq):
    grid = (B // BM,)
    return pl.pallas_call(
        _dist_kernel,
        grid=grid,
        in_specs=[
            pl.BlockSpec((BM, D), lambda i: (i, 0)),
            pl.BlockSpec((K, D), lambda i: (0, 0)),
            pl.BlockSpec((K,), lambda i: (0,)),
        ],
        out_specs=pl.BlockSpec((BM,), lambda i: (i,)),
        out_shape=jax.ShapeDtypeStruct((B,), jnp.int32),
        compiler_params=pltpu.CompilerParams(
            dimension_semantics=("arbitrary",),
        ),
    )(r, cb, csq)


def _tc_decode(q0, q1, q2, dec_W, dec_b):
    grid = (B // BM,)
    return pl.pallas_call(
        _decode_kernel,
        grid=grid,
        in_specs=[
            pl.BlockSpec((BM, D), lambda i: (i, 0)),
            pl.BlockSpec((BM, D), lambda i: (i, 0)),
            pl.BlockSpec((BM, D), lambda i: (i, 0)),
            pl.BlockSpec((D, D), lambda i: (0, 0)),
            pl.BlockSpec((1, D), lambda i: (0, 0)),
        ],
        out_specs=pl.BlockSpec((BM, D), lambda i: (i, 0)),
        out_shape=jax.ShapeDtypeStruct((B, D), jnp.float32),
        compiler_params=pltpu.CompilerParams(
            dimension_semantics=("arbitrary",),
        ),
    )(q0, q1, q2, dec_W, dec_b)


def _make_sc_gather():
    info = plsc.get_sparse_core_info()
    nc, ns = info.num_cores, info.num_subcores
    nw = nc * ns
    b_per_w = B // nw
    mesh = plsc.VectorSubcoreMesh(core_axis_name="c", subcore_axis_name="s")

    @functools.partial(
        pl.kernel,
        mesh=mesh,
        out_type=jax.ShapeDtypeStruct((B, D), jnp.float32),
        scratch_types=[
            pltpu.VMEM((b_per_w,), jnp.int32),
            pltpu.VMEM((b_per_w, D), jnp.float32),
            pltpu.SemaphoreType.DMA,
            pltpu.SemaphoreType.DMA,
            pltpu.SemaphoreType.DMA,
            pltpu.SemaphoreType.DMA,
        ],
    )
    def gather(table_hbm, idx_hbm, out_hbm, idx_v, rows_v, s0, s1, s2, s3):
        wid = lax.axis_index("s") * nc + lax.axis_index("c")
        base = wid * b_per_w
        h = b_per_w // 2
        pltpu.sync_copy(idx_hbm.at[pl.ds(base, b_per_w)], idx_v)
        g0 = pltpu.async_copy(
            table_hbm.at[idx_v.at[pl.ds(0, h)]], rows_v.at[pl.ds(0, h)], s0
        )
        g1 = pltpu.async_copy(
            table_hbm.at[idx_v.at[pl.ds(h, h)]], rows_v.at[pl.ds(h, h)], s1
        )
        g0.wait()
        w0 = pltpu.async_copy(
            rows_v.at[pl.ds(0, h)], out_hbm.at[pl.ds(base, h)], s2
        )
        g1.wait()
        w1 = pltpu.async_copy(
            rows_v.at[pl.ds(h, h)], out_hbm.at[pl.ds(base + h, h)], s3
        )
        w0.wait()
        w1.wait()

    return gather


def kernel(dense_content_embedding, enc_W, enc_b, cb0, cb1, cb2, dec_W, dec_b):
    sc_gather = _make_sc_gather()
    enc_b2 = enc_b.reshape(1, D)
    dec_b2 = dec_b.reshape(1, D)
    csq0 = jnp.sum(cb0 * cb0, axis=-1)[None, :]

    idx0, csq1 = _tc_enc_dist(dense_content_embedding, enc_W, enc_b2, cb0, csq0, cb1)
    q0 = sc_gather(cb0, idx0)
    idx1, csq2 = _tc_dist_csq(q0, cb1, csq1, cb2)
    q1 = sc_gather(cb1, idx1)
    idx2 = _tc_dist(q1, cb2, csq2)
    q2 = sc_gather(cb2, idx2)
    return _tc_decode(q0, q1, q2, dec_W, dec_b2)


# speedup vs baseline: 1.0220x; 1.0220x over previous
import functools

import jax
import jax.numpy as jnp
from jax import lax
from jax.experimental import pallas as pl
from jax.experimental.pallas import tpu as pltpu
from jax.experimental.pallas import tpu_sc as plsc

K = 8192
D = 256
IN_D = 768
B = 4096
BM = 1024

_PREC = None


def _enc_dist_kernel(x_ref, w_ref, b_ref, cb_ref, csq_ref, idx_ref):
    r0 = (
        lax.dot_general(
            x_ref[...], w_ref[...], (((1,), (0,)), ((), ())), precision=_PREC
        )
        + b_ref[...]
    )
    _argmin_body(r0, cb_ref, csq_ref, idx_ref)


def _dist_kernel(r_ref, cb_ref, csq_ref, idx_ref):
    _argmin_body(r_ref[...], cb_ref, csq_ref, idx_ref)


def _argmin_body(residual, cb_ref, csq_ref, idx_ref):
    r_sq = jnp.sum(residual * residual, axis=1, keepdims=True)
    cross2 = lax.dot_general(
        residual + residual, cb_ref[...], (((1,), (1,)), ((), ())), precision=_PREC
    )
    dist = r_sq - cross2 + csq_ref[...]
    m = jnp.min(dist, axis=1, keepdims=True)
    lane = lax.broadcasted_iota(jnp.int32, (1, K), 1).astype(jnp.float32)
    idxf = jnp.min(jnp.where(dist == m, lane, float(K)), axis=1)
    idx_ref[...] = idxf.astype(jnp.int32)


def _decode_kernel(q0_ref, q1_ref, q2_ref, w_ref, b_ref, out_ref):
    rec = q0_ref[...] + q1_ref[...] + q2_ref[...]
    out_ref[...] = (
        lax.dot_general(rec, w_ref[...], (((1,), (0,)), ((), ())), precision=_PREC)
        + b_ref[...]
    )


def _tc_enc_dist(x, enc_W, enc_b, cb, csq):
    grid = (B // BM,)
    return pl.pallas_call(
        _enc_dist_kernel,
        grid=grid,
        in_specs=[
            pl.BlockSpec((BM, IN_D), lambda i: (i, 0)),
            pl.BlockSpec((IN_D, D), lambda i: (0, 0)),
            pl.BlockSpec((1, D), lambda i: (0, 0)),
            pl.BlockSpec((K, D), lambda i: (0, 0)),
            pl.BlockSpec((1, K), lambda i: (0, 0)),
        ],
        out_specs=pl.BlockSpec((BM,), lambda i: (i,)),
        out_shape=jax.ShapeDtypeStruct((B,), jnp.int32),
        compiler_params=pltpu.CompilerParams(
            dimension_semantics=("arbitrary",),
        ),
    )(x, enc_W, enc_b, cb, csq)


def _tc_dist(r, cb, csq):
    grid = (B // BM,)
    return pl.pallas_call(
        _dist_kernel,
        grid=grid,
        in_specs=[
            pl.BlockSpec((BM, D), lambda i: (i, 0)),
            pl.BlockSpec((K, D), lambda i: (0, 0)),
            pl.BlockSpec((1, K), lambda i: (0, 0)),
        ],
        out_specs=pl.BlockSpec((BM,), lambda i: (i,)),
        out_shape=jax.ShapeDtypeStruct((B,), jnp.int32),
        compiler_params=pltpu.CompilerParams(
            dimension_semantics=("arbitrary",),
        ),
    )(r, cb, csq)


def _tc_decode(q0, q1, q2, dec_W, dec_b):
    grid = (B // BM,)
    return pl.pallas_call(
        _decode_kernel,
        grid=grid,
        in_specs=[
            pl.BlockSpec((BM, D), lambda i: (i, 0)),
            pl.BlockSpec((BM, D), lambda i: (i, 0)),
            pl.BlockSpec((BM, D), lambda i: (i, 0)),
            pl.BlockSpec((D, D), lambda i: (0, 0)),
            pl.BlockSpec((1, D), lambda i: (0, 0)),
        ],
        out_specs=pl.BlockSpec((BM, D), lambda i: (i, 0)),
        out_shape=jax.ShapeDtypeStruct((B, D), jnp.float32),
        compiler_params=pltpu.CompilerParams(
            dimension_semantics=("arbitrary",),
        ),
    )(q0, q1, q2, dec_W, dec_b)


def _make_sc_gather():
    info = plsc.get_sparse_core_info()
    nc, ns = info.num_cores, info.num_subcores
    nw = nc * ns
    b_per_w = B // nw
    mesh = plsc.VectorSubcoreMesh(core_axis_name="c", subcore_axis_name="s")

    @functools.partial(
        pl.kernel,
        mesh=mesh,
        out_type=jax.ShapeDtypeStruct((B, D), jnp.float32),
        scratch_types=[
            pltpu.VMEM((b_per_w,), jnp.int32),
            pltpu.VMEM((b_per_w, D), jnp.float32),
            pltpu.SemaphoreType.DMA,
            pltpu.SemaphoreType.DMA,
            pltpu.SemaphoreType.DMA,
            pltpu.SemaphoreType.DMA,
        ],
    )
    def gather(table_hbm, idx_hbm, out_hbm, idx_v, rows_v, s0, s1, s2, s3):
        wid = lax.axis_index("s") * nc + lax.axis_index("c")
        base = wid * b_per_w
        h = b_per_w // 2
        pltpu.sync_copy(idx_hbm.at[pl.ds(base, b_per_w)], idx_v)
        g0 = pltpu.async_copy(
            table_hbm.at[idx_v.at[pl.ds(0, h)]], rows_v.at[pl.ds(0, h)], s0
        )
        g1 = pltpu.async_copy(
            table_hbm.at[idx_v.at[pl.ds(h, h)]], rows_v.at[pl.ds(h, h)], s1
        )
        g0.wait()
        w0 = pltpu.async_copy(
            rows_v.at[pl.ds(0, h)], out_hbm.at[pl.ds(base, h)], s2
        )
        g1.wait()
        w1 = pltpu.async_copy(
            rows_v.at[pl.ds(h, h)], out_hbm.at[pl.ds(base + h, h)], s3
        )
        w0.wait()
        w1.wait()

    return gather


def kernel(dense_content_embedding, enc_W, enc_b, cb0, cb1, cb2, dec_W, dec_b):
    sc_gather = _make_sc_gather()
    enc_b2 = enc_b.reshape(1, D)
    dec_b2 = dec_b.reshape(1, D)
    csq0 = jnp.sum(cb0 * cb0, axis=-1)[None, :]
    csq1 = jnp.sum(cb1 * cb1, axis=-1)[None, :]
    csq2 = jnp.sum(cb2 * cb2, axis=-1)[None, :]

    idx0 = _tc_enc_dist(dense_content_embedding, enc_W, enc_b2, cb0, csq0)
    q0 = sc_gather(cb0, idx0)
    idx1 = _tc_dist(q0, cb1, csq1)
    q1 = sc_gather(cb1, idx1)
    idx2 = _tc_dist(q1, cb2, csq2)
    q2 = sc_gather(cb2, idx2)
    return _tc_decode(q0, q1, q2, dec_W, dec_b2)
